# baseline (speedup 1.0000x reference)
import jax
import jax.numpy as jnp
from jax import lax
from jax.experimental import pallas as pl
from jax.experimental.pallas import tpu as pltpu


def _pick_block(n):
    for t in (4096, 2048, 1024, 512, 256, 128, 64, 32, 16, 8):
        if n % t == 0:
            return t
    return n


def kernel(feats, offsets, ln_g, ln_b, W1, b1, W2, b2):
    N, D = feats.shape
    B = offsets.shape[0] - 1
    H = W1.shape[1]
    T = _pick_block(N)
    K = N // T

    lng_col = ln_g.reshape(D, 1)
    lnb_row = ln_b.reshape(1, D)
    b1_row = b1.reshape(1, H)
    W2_row = W2.reshape(1, H)
    b2_r = b2.reshape(1, 1)
    offs = offsets.astype(jnp.int32)

    inv_s2 = 0.7071067811865476

    def body(x_ref, offs_ref, lng_ref, lnb_ref, W1_ref, b1_ref, W2_ref,
             b2_ref, out_ref, m_ref, s_ref, acc_ref,
             Wg_ref, cs_ref, c1_ref, W2r_ref, st_ref, en_ref):
        i = pl.program_id(0)

        @pl.when(i == 0)
        def _init():
            m_ref[...] = jnp.full_like(m_ref, -jnp.inf)
            s_ref[...] = jnp.zeros_like(s_ref)
            acc_ref[...] = jnp.zeros_like(acc_ref)
            W1v = W1_ref[...]
            Wg0 = lng_ref[...] * W1v * inv_s2
            Wg_ref[...] = Wg0
            cs_ref[...] = jnp.sum(Wg0, axis=0, keepdims=True)
            c1_ref[...] = (jnp.dot(lnb_ref[...], W1v,
                                   preferred_element_type=jnp.float32)
                           + b1_ref[...]) * inv_s2
            W2r_ref[...] = W2_ref[...] * inv_s2
            bidx = lax.broadcasted_iota(jnp.int32, (B, 1), 0)
            st0 = jnp.zeros((B, 1), jnp.int32)
            en0 = jnp.zeros((B, 1), jnp.int32)
            for b in range(B):
                st0 = jnp.where(bidx == b, offs_ref[b], st0)
                en0 = jnp.where(bidx == b, offs_ref[b + 1], en0)
            st_ref[...] = st0
            en_ref[...] = en0

        Wg = Wg_ref[...]
        csum = cs_ref[...]
        c1 = c1_ref[...]
        W2r = W2r_ref[...]
        st = st_ref[...]
        en = en_ref[...]

        x = x_ref[...]
        mu = jnp.mean(x, axis=1, keepdims=True)
        ms = jnp.mean(x * x, axis=1, keepdims=True)
        var = jnp.maximum(ms - mu * mu, 0.0)
        r = lax.rsqrt(var + 1e-5)
        xw = jnp.dot(x, Wg,
                     preferred_element_type=jnp.float32)
        u = r * xw - (r * mu) * csum + c1
        h = u * (1.0 + lax.erf(u))
        w_row = lax.dot_general(W2r, h, (((1,), (1,)), ((), ())),
                                preferred_element_type=jnp.float32)
        w_row = w_row + b2_ref[...]

        gidx = i * T + lax.broadcasted_iota(jnp.int32, (B, T), 1)
        mask = (gidx >= st) & (gidx < en)

        wneg = jnp.where(mask, w_row, -jnp.inf)
        bmax = jnp.max(wneg, axis=1, keepdims=True)
        m_old = m_ref[...]
        m_new = jnp.maximum(m_old, bmax)
        m_safe = jnp.where(m_new > -jnp.inf, m_new, 0.0)
        e = jnp.exp(wneg - m_safe)
        scale = jnp.where(m_old > -jnp.inf, jnp.exp(m_old - m_new), 0.0)

        s_ref[...] = s_ref[...] * scale + jnp.sum(e, axis=1, keepdims=True)
        acc_ref[...] = acc_ref[...] * scale + jnp.dot(
            e, x, preferred_element_type=jnp.float32)
        m_ref[...] = m_new

        @pl.when(i == pl.num_programs(0) - 1)
        def _fin():
            s = s_ref[...]
            out_ref[...] = acc_ref[...] / jnp.where(s > 0, s, 1.0)

    out = pl.pallas_call(
        body,
        grid=(K,),
        in_specs=[
            pl.BlockSpec((T, D), lambda i: (i, 0)),
            pl.BlockSpec(memory_space=pltpu.SMEM),
            pl.BlockSpec((D, 1), lambda i: (0, 0)),
            pl.BlockSpec((1, D), lambda i: (0, 0)),
            pl.BlockSpec((D, H), lambda i: (0, 0)),
            pl.BlockSpec((1, H), lambda i: (0, 0)),
            pl.BlockSpec((1, H), lambda i: (0, 0)),
            pl.BlockSpec((1, 1), lambda i: (0, 0)),
        ],
        out_specs=pl.BlockSpec((B, D), lambda i: (0, 0)),
        out_shape=jax.ShapeDtypeStruct((B, D), jnp.float32),
        scratch_shapes=[
            pltpu.VMEM((B, 1), jnp.float32),
            pltpu.VMEM((B, 1), jnp.float32),
            pltpu.VMEM((B, D), jnp.float32),
            pltpu.VMEM((D, H), jnp.float32),
            pltpu.VMEM((1, H), jnp.float32),
            pltpu.VMEM((1, H), jnp.float32),
            pltpu.VMEM((1, H), jnp.float32),
            pltpu.VMEM((B, 1), jnp.int32),
            pltpu.VMEM((B, 1), jnp.int32),
        ],
    )(feats, offs, lng_col, lnb_row, W1, b1_row, W2_row, b2_r)
    return out


# speedup vs baseline: 1.0891x; 1.0891x over previous
import jax
import jax.numpy as jnp
from jax import lax
from jax.experimental import pallas as pl
from jax.experimental.pallas import tpu as pltpu


def _pick_block(n):
    for t in (8192, 4096, 2048, 1024, 512, 256, 128, 64, 32, 16, 8):
        if n % t == 0:
            return t
    return n


def kernel(feats, offsets, ln_g, ln_b, W1, b1, W2, b2):
    N, D = feats.shape
    B = offsets.shape[0] - 1
    H = W1.shape[1]
    T = _pick_block(N)
    K = N // T

    lng_col = ln_g.reshape(D, 1)
    lnb_row = ln_b.reshape(1, D)
    b1_row = b1.reshape(1, H)
    W2_row = W2.reshape(1, H)
    b2_r = b2.reshape(1, 1)
    offs = offsets.astype(jnp.int32)

    inv_s2 = 0.7071067811865476

    def body(x_ref, offs_ref, lng_ref, lnb_ref, W1_ref, b1_ref, W2_ref,
             b2_ref, out_ref, m_ref, s_ref, acc_ref,
             Wg_ref, c1_ref, W2r_ref, st_ref, en_ref):
        i = pl.program_id(0)

        @pl.when(i == 0)
        def _init():
            m_ref[...] = jnp.full_like(m_ref, -jnp.inf)
            s_ref[...] = jnp.zeros_like(s_ref)
            acc_ref[...] = jnp.zeros_like(acc_ref)
            W1v = W1_ref[...]
            Wg0 = lng_ref[...] * W1v * inv_s2
            cs0 = jnp.sum(Wg0, axis=0, keepdims=True)
            Wg_ref[...] = Wg0 - cs0 * (1.0 / D)
            c1_ref[...] = (jnp.dot(lnb_ref[...], W1v,
                                   preferred_element_type=jnp.float32)
                           + b1_ref[...]) * inv_s2
            W2r_ref[...] = W2_ref[...] * inv_s2
            bidx = lax.broadcasted_iota(jnp.int32, (B, 1), 0)
            st0 = jnp.zeros((B, 1), jnp.int32)
            en0 = jnp.zeros((B, 1), jnp.int32)
            for b in range(B):
                st0 = jnp.where(bidx == b, offs_ref[b], st0)
                en0 = jnp.where(bidx == b, offs_ref[b + 1], en0)
            st_ref[...] = st0
            en_ref[...] = en0

        Wg = Wg_ref[...]
        c1 = c1_ref[...]
        W2r = W2r_ref[...]
        st = st_ref[...]
        en = en_ref[...]

        x = x_ref[...]
        mu = jnp.mean(x, axis=1, keepdims=True)
        ms = jnp.mean(x * x, axis=1, keepdims=True)
        var = jnp.maximum(ms - mu * mu, 0.0)
        r = lax.rsqrt(var + 1e-5)
        xw = jnp.dot(x, Wg,
                     preferred_element_type=jnp.float32)
        u = r * xw + c1
        h = u * (1.0 + lax.erf(u))
        w_row = lax.dot_general(W2r, h, (((1,), (1,)), ((), ())),
                                preferred_element_type=jnp.float32)
        w_row = w_row + b2_ref[...]

        gidx = i * T + lax.broadcasted_iota(jnp.int32, (B, T), 1)
        mask = (gidx >= st) & (gidx < en)

        wneg = jnp.where(mask, w_row, -jnp.inf)
        bmax = jnp.max(wneg, axis=1, keepdims=True)
        m_old = m_ref[...]
        m_new = jnp.maximum(m_old, bmax)
        m_safe = jnp.where(m_new > -jnp.inf, m_new, 0.0)
        e = jnp.exp(wneg - m_safe)
        scale = jnp.where(m_old > -jnp.inf, jnp.exp(m_old - m_new), 0.0)

        s_ref[...] = s_ref[...] * scale + jnp.sum(e, axis=1, keepdims=True)
        acc_ref[...] = acc_ref[...] * scale + jnp.dot(
            e, x, preferred_element_type=jnp.float32)
        m_ref[...] = m_new

        @pl.when(i == pl.num_programs(0) - 1)
        def _fin():
            s = s_ref[...]
            out_ref[...] = acc_ref[...] / jnp.where(s > 0, s, 1.0)

    out = pl.pallas_call(
        body,
        grid=(K,),
        in_specs=[
            pl.BlockSpec((T, D), lambda i: (i, 0)),
            pl.BlockSpec(memory_space=pltpu.SMEM),
            pl.BlockSpec((D, 1), lambda i: (0, 0)),
            pl.BlockSpec((1, D), lambda i: (0, 0)),
            pl.BlockSpec((D, H), lambda i: (0, 0)),
            pl.BlockSpec((1, H), lambda i: (0, 0)),
            pl.BlockSpec((1, H), lambda i: (0, 0)),
            pl.BlockSpec((1, 1), lambda i: (0, 0)),
        ],
        out_specs=pl.BlockSpec((B, D), lambda i: (0, 0)),
        out_shape=jax.ShapeDtypeStruct((B, D), jnp.float32),
        scratch_shapes=[
            pltpu.VMEM((B, 1), jnp.float32),
            pltpu.VMEM((B, 1), jnp.float32),
            pltpu.VMEM((B, D), jnp.float32),
            pltpu.VMEM((D, H), jnp.float32),
            pltpu.VMEM((1, H), jnp.float32),
            pltpu.VMEM((1, H), jnp.float32),
            pltpu.VMEM((B, 1), jnp.int32),
            pltpu.VMEM((B, 1), jnp.int32),
        ],
    )(feats, offs, lng_col, lnb_row, W1, b1_row, W2_row, b2_r)
    return out
